# Initial kernel scaffold; baseline (speedup 1.0000x reference)
import jax
import jax.numpy as jnp
from jax.experimental import pallas as pl


def kernel(X):
    raise NotImplementedError("write your pallas kernel here")


# speedup vs baseline: 20.5701x; 20.5701x over previous
import functools

import jax
import jax.numpy as jnp
from jax import lax
from jax.experimental import pallas as pl
from jax.experimental.pallas import tpu as pltpu
from jax.experimental.pallas import tpu_sc as plsc

ROWS = 128
T_IN = 122880
T_OUT = 23552

NUM_CORES = 2
NUM_SUBCORES = 16
NUM_WORKERS = NUM_CORES * NUM_SUBCORES
ROWS_PER_WORKER = ROWS // NUM_WORKERS

STRIDED_SEGS = (
    (0, 8, 0, 5, 16384, 2048),
    (81920, 4, 10240, 3, 12288, 3072),
)
COPY_SEG = (118784, 19456, 4096)

IN_BUF = 16384
OUT_BUF = 3072


def _decimator_body(x_hbm, out_hbm, in_v, out_v):
  cid = lax.axis_index("c")
  sid = lax.axis_index("s")
  wid = cid * NUM_SUBCORES + sid

  lanes = lax.iota(jnp.int32, 16)

  for k in range(ROWS_PER_WORKER):
    r = wid * ROWS_PER_WORKER + k

    for (in_off, stride, out_off, n_chunks, in_chunk, out_chunk) in STRIDED_SEGS:
      idx0 = lanes * stride

      def chunk_body(c, carry, in_off=in_off, stride=stride, out_off=out_off,
                     in_chunk=in_chunk, out_chunk=out_chunk, idx0=idx0, r=r):
        pltpu.sync_copy(
            x_hbm.at[r, pl.ds(in_off + c * in_chunk, in_chunk)],
            in_v.at[pl.ds(0, in_chunk)],
        )

        def gather_body(j, carry2, idx0=idx0, stride=stride):
          idx = idx0 + j * (16 * stride)
          vals = plsc.load_gather(in_v, [idx])
          out_v[pl.ds(j * 16, 16)] = vals
          return carry2

        lax.fori_loop(0, out_chunk // 16, gather_body, 0, unroll=4)

        pltpu.sync_copy(
            out_v.at[pl.ds(0, out_chunk)],
            out_hbm.at[r, pl.ds(out_off + c * out_chunk, out_chunk)],
        )
        return carry

      lax.fori_loop(0, n_chunks, chunk_body, 0)

    in_off, out_off, length = COPY_SEG
    pltpu.sync_copy(
        x_hbm.at[r, pl.ds(in_off, length)],
        out_hbm.at[r, pl.ds(out_off, length)],
    )


@jax.jit
def _decimate(x2d):
  mesh = plsc.VectorSubcoreMesh(core_axis_name="c", subcore_axis_name="s")
  f = functools.partial(
      pl.kernel,
      mesh=mesh,
      out_type=jax.ShapeDtypeStruct((ROWS, T_OUT), jnp.float32),
      scratch_types=[
          pltpu.VMEM((IN_BUF,), jnp.float32),
          pltpu.VMEM((OUT_BUF,), jnp.float32),
      ],
      compiler_params=pltpu.CompilerParams(needs_layout_passes=False),
  )(_decimator_body)
  return f(x2d)


def kernel(X):
  assert X.shape == (16, 8, T_IN), X.shape
  x2d = X.reshape(ROWS, T_IN)
  out = _decimate(x2d)
  return out.reshape(16, 8, T_OUT)
